# baseline (speedup 1.0000x reference)
import jax
import jax.numpy as jnp
from jax.experimental import pallas as pl

_N, _D, _H, _E = 16384, 1024, 128, 16
_S = 8
_R = 256
_STEPS = _N // (_S * _R)


def _mlp_onehot_body(*refs):
    x_refs = refs[:_S]
    w1_ref, b1_ref, w2_ref, b2_ref, out_ref = refs[_S:]
    xcat = jnp.concatenate([r[...] for r in x_refs], axis=0)
    acc = jnp.dot(xcat[:, 0:256], w1_ref[0:256, :],
                  preferred_element_type=jnp.float32)
    for k in range(1, 4):
        acc = acc + jnp.dot(xcat[:, k * 256:(k + 1) * 256],
                            w1_ref[k * 256:(k + 1) * 256, :],
                            preferred_element_type=jnp.float32)
    h = jnp.maximum(acc + b1_ref[...], 0.0)
    logits = jnp.dot(h, w2_ref[...], preferred_element_type=jnp.float32)
    logits = logits + b2_ref[...]
    m = jnp.max(logits, axis=-1, keepdims=True)
    e = jnp.exp(logits - m)
    s = jnp.sum(e, axis=-1, keepdims=True)
    y = e / s
    ym = 1.0 / s
    eq = (y == ym).astype(jnp.float32)
    row_i = jax.lax.broadcasted_iota(jnp.int32, (_E, _E), 0)
    col_i = jax.lax.broadcasted_iota(jnp.int32, (_E, _E), 1)
    strict_upper = (row_i < col_i).astype(jnp.float32)
    cum = jnp.dot(eq, strict_upper, preferred_element_type=jnp.float32)
    out_ref[...] = jnp.where(cum == 0.0, eq, 0.0)


def kernel(x, W1, b1, W2, b2):
    return pl.pallas_call(
        _mlp_onehot_body,
        grid=(_STEPS,),
        in_specs=[
            pl.BlockSpec((_R, _D), lambda i, j=j: (_S * i + j, 0))
            for j in range(_S)
        ] + [
            pl.BlockSpec((_D, _H), lambda i: (0, 0)),
            pl.BlockSpec((1, _H), lambda i: (0, 0)),
            pl.BlockSpec((_H, _E), lambda i: (0, 0)),
            pl.BlockSpec((1, _E), lambda i: (0, 0)),
        ],
        out_specs=pl.BlockSpec((_S * _R, _E), lambda i: (i, 0)),
        out_shape=jax.ShapeDtypeStruct((_N, _E), jnp.float32),
    )(*([x] * _S), W1, b1.reshape(1, _H), W2, b2.reshape(1, _E))


# speedup vs baseline: 1.0154x; 1.0154x over previous
import jax
import jax.numpy as jnp
from jax.experimental import pallas as pl

_N, _D, _H, _E = 16384, 1024, 128, 16
_S = 8
_R = 256
_STEPS = _N // (_S * _R)


def _mlp_onehot_body(*refs):
    x_refs = refs[:_S]
    w1_ref, b1_ref, w2_ref, b2_ref, out_ref = refs[_S:]
    xcat = jnp.concatenate([r[...] for r in x_refs], axis=0)
    acc = jnp.dot(xcat[:, 0:256], w1_ref[0:256, :],
                  preferred_element_type=jnp.float32)
    for k in range(1, 4):
        acc = acc + jnp.dot(xcat[:, k * 256:(k + 1) * 256],
                            w1_ref[k * 256:(k + 1) * 256, :],
                            preferred_element_type=jnp.float32)
    h = jnp.maximum(acc + b1_ref[...], 0.0)
    row_i = jax.lax.broadcasted_iota(jnp.int32, (_E, _E), 0)
    col_i = jax.lax.broadcasted_iota(jnp.int32, (_E, _E), 1)
    strict_upper = (row_i < col_i).astype(jnp.float32)
    rows = _S * _R
    for j in range(rows // 256):
        hc = h[j * 256:(j + 1) * 256, :]
        logits = jnp.dot(hc, w2_ref[...], preferred_element_type=jnp.float32)
        logits = logits + b2_ref[...]
        m = jnp.max(logits, axis=-1, keepdims=True)
        e = jnp.exp(logits - m)
        s = jnp.sum(e, axis=-1, keepdims=True)
        y = e / s
        ym = 1.0 / s
        eq = (y == ym).astype(jnp.float32)
        cum = jnp.dot(eq, strict_upper, preferred_element_type=jnp.float32)
        out_ref[j * 256:(j + 1) * 256, :] = jnp.where(cum == 0.0, eq, 0.0)


def kernel(x, W1, b1, W2, b2):
    return pl.pallas_call(
        _mlp_onehot_body,
        grid=(_STEPS,),
        in_specs=[
            pl.BlockSpec((_R, _D), lambda i, j=j: (_S * i + j, 0))
            for j in range(_S)
        ] + [
            pl.BlockSpec((_D, _H), lambda i: (0, 0)),
            pl.BlockSpec((1, _H), lambda i: (0, 0)),
            pl.BlockSpec((_H, _E), lambda i: (0, 0)),
            pl.BlockSpec((1, _E), lambda i: (0, 0)),
        ],
        out_specs=pl.BlockSpec((_S * _R, _E), lambda i: (i, 0)),
        out_shape=jax.ShapeDtypeStruct((_N, _E), jnp.float32),
    )(*([x] * _S), W1, b1.reshape(1, _H), W2, b2.reshape(1, _E))


# speedup vs baseline: 1.0276x; 1.0120x over previous
import jax
import jax.numpy as jnp
from jax.experimental import pallas as pl

_N, _D, _H, _E = 16384, 1024, 128, 16
_S = 1
_R = 2048
_STEPS = _N // (_S * _R)


def _mlp_onehot_body(*refs):
    x_refs = refs[:_S]
    w1_ref, b1_ref, w2_ref, b2_ref, out_ref = refs[_S:]
    xcat = jnp.concatenate([r[...] for r in x_refs], axis=0)
    acc = jnp.dot(xcat[:, 0:256], w1_ref[0:256, :],
                  preferred_element_type=jnp.float32)
    for k in range(1, 4):
        acc = acc + jnp.dot(xcat[:, k * 256:(k + 1) * 256],
                            w1_ref[k * 256:(k + 1) * 256, :],
                            preferred_element_type=jnp.float32)
    h = jnp.maximum(acc + b1_ref[...], 0.0)
    row_i = jax.lax.broadcasted_iota(jnp.int32, (_E, _E), 0)
    col_i = jax.lax.broadcasted_iota(jnp.int32, (_E, _E), 1)
    strict_upper = (row_i < col_i).astype(jnp.float32)
    rows = _S * _R
    for j in range(rows // 256):
        hc = h[j * 256:(j + 1) * 256, :]
        logits = jnp.dot(hc, w2_ref[...], preferred_element_type=jnp.float32)
        logits = logits + b2_ref[...]
        m = jnp.max(logits, axis=-1, keepdims=True)
        e = jnp.exp(logits - m)
        s = jnp.sum(e, axis=-1, keepdims=True)
        y = e / s
        ym = 1.0 / s
        eq = (y == ym).astype(jnp.float32)
        cum = jnp.dot(eq, strict_upper, preferred_element_type=jnp.float32)
        out_ref[j * 256:(j + 1) * 256, :] = jnp.where(cum == 0.0, eq, 0.0)


def kernel(x, W1, b1, W2, b2):
    return pl.pallas_call(
        _mlp_onehot_body,
        grid=(_STEPS,),
        in_specs=[
            pl.BlockSpec((_R, _D), lambda i, j=j: (_S * i + j, 0))
            for j in range(_S)
        ] + [
            pl.BlockSpec((_D, _H), lambda i: (0, 0)),
            pl.BlockSpec((1, _H), lambda i: (0, 0)),
            pl.BlockSpec((_H, _E), lambda i: (0, 0)),
            pl.BlockSpec((1, _E), lambda i: (0, 0)),
        ],
        out_specs=pl.BlockSpec((_S * _R, _E), lambda i: (i, 0)),
        out_shape=jax.ShapeDtypeStruct((_N, _E), jnp.float32),
    )(*([x] * _S), W1, b1.reshape(1, _H), W2, b2.reshape(1, _E))
